# Initial kernel scaffold; baseline (speedup 1.0000x reference)
import jax
import jax.numpy as jnp
from jax.experimental import pallas as pl


def kernel(x, edge_index, gcn_w, gcn_b, sage_wl, sage_bl, sage_wr, w1, b1, w2, b2, w3, b3, vw, vb):
    raise NotImplementedError("write your pallas kernel here")


# speedup vs baseline: 10.7303x; 10.7303x over previous
import functools
import math

import jax
import jax.numpy as jnp
from jax import lax
from jax.experimental import pallas as pl
from jax.experimental.pallas import tpu as pltpu
from jax.experimental.pallas import tpu_sc as plsc

N = 10000
E = 320000
H = 128
OUT = 8

NC = 2
NS = 16
NW = NC * NS
EPT = E // NW
CH = 80
NCHUNK = EPT // CH
RPT = N // NS
ZR = 125

_mesh = plsc.VectorSubcoreMesh(core_axis_name="c", subcore_axis_name="s")


def _zero_f32_ref(ref, nwords):
    z = jnp.zeros((16,), jnp.float32)

    def body(i, _):
        ref[pl.ds(i * 16, 16)] = z
        return 0

    lax.fori_loop(0, nwords // 16, body, 0)


def _zero_f32_ref2d(ref, rows, cols):
    z = jnp.zeros((16,), jnp.float32)

    def body(i, _):
        for j in range(cols // 16):
            ref[i, pl.ds(j * 16, 16)] = z
        return 0

    lax.fori_loop(0, rows, body, 0)


@functools.partial(
    pl.kernel,
    out_type=jax.ShapeDtypeStruct((NC, N), jnp.float32),
    mesh=_mesh,
    scratch_types=[
        pltpu.VMEM((CH,), jnp.int32),
        pltpu.VMEM((CH,), jnp.float32),
        pltpu.VMEM((1008,), jnp.float32),
        pltpu.VMEM_SHARED((N,), jnp.float32),
    ],
    compiler_params=pltpu.CompilerParams(use_tc_tiling_on_sc=False),
)
def _sc_count(dst, out, idx_v, ones_v, zb, cnt_sp):
    cid = lax.axis_index("c")
    sid = lax.axis_index("s")
    wid = cid * NS + sid

    one = jnp.ones((16,), jnp.float32)
    for j in range(CH // 16):
        ones_v[pl.ds(j * 16, 16)] = one
    _zero_f32_ref(zb, 1008)

    @pl.when(sid < 10)
    def _():
        pltpu.sync_copy(zb.at[pl.ds(0, 1000)], cnt_sp.at[pl.ds(sid * 1000, 1000)])

    plsc.subcore_barrier()

    base = wid * EPT

    def chunk(i, _):
        pltpu.sync_copy(dst.at[pl.ds(base + i * CH, CH)], idx_v)
        pltpu.sync_copy(ones_v, cnt_sp.at[idx_v], add=True)
        return 0

    lax.fori_loop(0, NCHUNK, chunk, 0)
    plsc.subcore_barrier()

    @pl.when(sid < 10)
    def _():
        pltpu.sync_copy(cnt_sp.at[pl.ds(sid * 1000, 1000)],
                        out.at[cid, pl.ds(sid * 1000, 1000)])


@functools.partial(
    pl.kernel,
    out_type=jax.ShapeDtypeStruct((NC, N, H), jnp.float32),
    mesh=_mesh,
    scratch_types=[
        pltpu.VMEM((CH,), jnp.int32),
        pltpu.VMEM((CH,), jnp.int32),
        pltpu.VMEM((CH, H), jnp.float32),
        pltpu.VMEM((ZR, H), jnp.float32),
        pltpu.VMEM_SHARED((N, H), jnp.float32),
        pltpu.SemaphoreType.DMA,
    ],
    compiler_params=pltpu.CompilerParams(use_tc_tiling_on_sc=False),
)
def _sc_agg(table, src, dst, out, src_v, dst_v, rows_v, zb, acc, sem):
    cid = lax.axis_index("c")
    sid = lax.axis_index("s")
    wid = cid * NS + sid

    _zero_f32_ref2d(zb, ZR, H)
    for j in range(RPT // ZR):
        pltpu.sync_copy(zb, acc.at[pl.ds(sid * RPT + j * ZR, ZR)])
    plsc.subcore_barrier()

    base = wid * EPT

    def chunk(i, _):
        off = base + i * CH
        pltpu.sync_copy(src.at[pl.ds(off, CH)], src_v)
        pltpu.sync_copy(dst.at[pl.ds(off, CH)], dst_v)
        pltpu.async_copy(table.at[src_v], rows_v, sem).wait()
        pltpu.sync_copy(rows_v, acc.at[dst_v], add=True)
        return 0

    lax.fori_loop(0, NCHUNK, chunk, 0)
    plsc.subcore_barrier()
    pltpu.sync_copy(acc.at[pl.ds(sid * RPT, RPT)],
                    out.at[cid, pl.ds(sid * RPT, RPT)])


def _dot(a, b):
    return jnp.dot(a, b, preferred_element_type=jnp.float32,
                   precision=lax.Precision.HIGHEST)


def _tc1_body(cnt2_ref, x_ref, w_ref, y_ref):
    cnt = cnt2_ref[0, :] + cnt2_ref[1, :]
    dinv = lax.rsqrt(cnt + 1.0)
    y_ref[...] = _dot(x_ref[...], w_ref[...]) * dinv[:, None]


def _tc2_body(agg2_ref, y_ref, cnt2_ref, b_ref, h_ref):
    cnt = cnt2_ref[0, :] + cnt2_ref[1, :]
    dinv = lax.rsqrt(cnt + 1.0)
    s = agg2_ref[0] + agg2_ref[1] + y_ref[...]
    h_ref[...] = jnp.maximum(s * dinv[:, None] + b_ref[...], 0.0)


def _gelu(v):
    return 0.5 * v * (1.0 + lax.erf(v * (1.0 / math.sqrt(2.0))))


def _tc3_body(aggs2_ref, cnt2_ref, h_ref, wl_ref, bl_ref, wr_ref,
              w1_ref, b1_ref, w2_ref, b2_ref, w3_ref, b3_ref,
              vw_ref, vb_ref, means_ref, values_ref):
    cnt = cnt2_ref[0, :] + cnt2_ref[1, :]
    inv = 1.0 / jnp.maximum(cnt, 1.0)
    mean = (aggs2_ref[0] + aggs2_ref[1]) * inv[:, None]
    h = h_ref[...]
    h2 = jnp.maximum(_dot(mean, wl_ref[...]) + bl_ref[...]
                     + _dot(h, wr_ref[...]), 0.0)
    a = _gelu(_dot(h2, w1_ref[...]) + b1_ref[...])
    a = _gelu(_dot(a, w2_ref[...]) + b2_ref[...])
    means_ref[...] = _dot(a, w3_ref[...]) + b3_ref[...]
    values_ref[...] = _dot(h2, vw_ref[...]) + vb_ref[...]


def _tc1(cnt2, x, gcn_w):
    return pl.pallas_call(
        _tc1_body,
        out_shape=jax.ShapeDtypeStruct((N, H), jnp.float32),
    )(cnt2, x, gcn_w)


def _tc2(agg2, y, cnt2, gcn_b):
    return pl.pallas_call(
        _tc2_body,
        out_shape=jax.ShapeDtypeStruct((N, H), jnp.float32),
    )(agg2, y, cnt2, gcn_b)


def _tc3(aggs2, cnt2, h, sage_wl, sage_bl, sage_wr, w1, b1, w2, b2, w3, b3,
         vw, vb):
    return pl.pallas_call(
        _tc3_body,
        out_shape=(jax.ShapeDtypeStruct((N, OUT), jnp.float32),
                   jax.ShapeDtypeStruct((N, 1), jnp.float32)),
        compiler_params=pltpu.CompilerParams(vmem_limit_bytes=64 * 1024 * 1024),
    )(aggs2, cnt2, h, sage_wl, sage_bl, sage_wr, w1, b1, w2, b2, w3, b3,
      vw, vb)


def kernel(x, edge_index, gcn_w, gcn_b, sage_wl, sage_bl, sage_wr,
           w1, b1, w2, b2, w3, b3, vw, vb):
    src = edge_index[0]
    dst = edge_index[1]
    cnt2 = _sc_count(dst)
    y = _tc1(cnt2, x, gcn_w)
    agg2 = _sc_agg(y, src, dst)
    h = _tc2(agg2, y, cnt2, gcn_b.reshape(1, H))
    aggs2 = _sc_agg(h, src, dst)
    means, values = _tc3(aggs2, cnt2, h, sage_wl, sage_bl.reshape(1, H),
                         sage_wr, w1, b1.reshape(1, H), w2, b2.reshape(1, H),
                         w3, b3.reshape(1, OUT), vw, vb.reshape(1, 1))
    return means, values.reshape(N)


# speedup vs baseline: 20.1034x; 1.8735x over previous
import functools
import math

import jax
import jax.numpy as jnp
from jax import lax
from jax.experimental import pallas as pl
from jax.experimental.pallas import tpu as pltpu
from jax.experimental.pallas import tpu_sc as plsc

N = 10000
E = 320000
H = 128
OUT = 8

NC = 2
NS = 16
NW = NC * NS
EPT = E // NW
CH = 80
NCHUNK = EPT // CH
RPT = N // NS
ZR = 25

_mesh = plsc.VectorSubcoreMesh(core_axis_name="c", subcore_axis_name="s")


def _zero_f32_ref(ref, nwords):
    z = jnp.zeros((16,), jnp.float32)

    def body(i, _):
        ref[pl.ds(i * 16, 16)] = z
        return 0

    lax.fori_loop(0, nwords // 16, body, 0)


def _zero_f32_ref2d(ref, rows, cols):
    z = jnp.zeros((16,), jnp.float32)

    def body(i, _):
        for j in range(cols // 16):
            ref[i, pl.ds(j * 16, 16)] = z
        return 0

    lax.fori_loop(0, rows, body, 0)


@functools.partial(
    pl.kernel,
    out_type=jax.ShapeDtypeStruct((NC, N), jnp.float32),
    mesh=_mesh,
    scratch_types=[
        pltpu.VMEM((NCHUNK, CH), jnp.int32),
        pltpu.VMEM((CH,), jnp.float32),
        pltpu.VMEM((1008,), jnp.float32),
        pltpu.VMEM_SHARED((N,), jnp.float32),
        pltpu.SemaphoreType.DMA,
        pltpu.SemaphoreType.DMA,
    ],
    compiler_params=pltpu.CompilerParams(use_tc_tiling_on_sc=False),
)
def _sc_count(dst2d, out, dst_v, ones_v, zb, cnt_sp, sem_i, sem_s):
    cid = lax.axis_index("c")
    sid = lax.axis_index("s")
    wid = cid * NS + sid

    idx_cp = pltpu.async_copy(dst2d.at[pl.ds(wid * NCHUNK, NCHUNK)], dst_v,
                              sem_i)
    one = jnp.ones((16,), jnp.float32)
    for j in range(CH // 16):
        ones_v[pl.ds(j * 16, 16)] = one
    _zero_f32_ref(zb, 1008)

    @pl.when(sid < 10)
    def _():
        pltpu.sync_copy(zb.at[pl.ds(0, 1000)], cnt_sp.at[pl.ds(sid * 1000, 1000)])

    idx_cp.wait()
    plsc.subcore_barrier()

    G = 5

    def group(g, _):
        for j in range(G):
            pltpu.async_copy(ones_v, cnt_sp.at[dst_v.at[g * G + j]], sem_s,
                             add=True)
        for j in range(G):
            pltpu.make_async_copy(ones_v, cnt_sp.at[dst_v.at[g * G + j]],
                                  sem_s).wait()
        return 0

    lax.fori_loop(0, NCHUNK // G, group, 0)
    plsc.subcore_barrier()

    @pl.when(sid < 10)
    def _():
        pltpu.sync_copy(cnt_sp.at[pl.ds(sid * 1000, 1000)],
                        out.at[cid, pl.ds(sid * 1000, 1000)])


@functools.partial(
    pl.kernel,
    out_type=jax.ShapeDtypeStruct((NC, N, H), jnp.float32),
    mesh=_mesh,
    scratch_types=[
        pltpu.VMEM((NCHUNK, CH), jnp.int32),
        pltpu.VMEM((NCHUNK, CH), jnp.int32),
        pltpu.VMEM((CH, H), jnp.float32),
        pltpu.VMEM((CH, H), jnp.float32),
        pltpu.VMEM((ZR, H), jnp.float32),
        pltpu.VMEM_SHARED((N, H), jnp.float32),
        pltpu.SemaphoreType.DMA,
        pltpu.SemaphoreType.DMA,
        pltpu.SemaphoreType.DMA,
        pltpu.SemaphoreType.DMA,
        pltpu.SemaphoreType.DMA,
    ],
    compiler_params=pltpu.CompilerParams(use_tc_tiling_on_sc=False),
)
def _sc_agg(table, src2d, dst2d, out, src_v, dst_v, ra, rb, zb, acc,
            sem_i, sem_ga, sem_gb, sem_sa, sem_sb):
    cid = lax.axis_index("c")
    sid = lax.axis_index("s")
    wid = cid * NS + sid
    row0 = wid * NCHUNK

    ia = pltpu.async_copy(src2d.at[pl.ds(row0, NCHUNK)], src_v, sem_i)
    ib = pltpu.async_copy(dst2d.at[pl.ds(row0, NCHUNK)], dst_v, sem_i)
    _zero_f32_ref2d(zb, ZR, H)
    ia.wait()
    ib.wait()
    for j in range(RPT // ZR):
        pltpu.sync_copy(zb, acc.at[pl.ds(sid * RPT + j * ZR, ZR)])
    pltpu.async_copy(table.at[src_v.at[0]], ra, sem_ga)
    pltpu.async_copy(table.at[src_v.at[1]], rb, sem_gb)
    plsc.subcore_barrier()

    def pair(k, _):
        c0 = 2 * k
        pltpu.make_async_copy(table.at[src_v.at[c0]], ra, sem_ga).wait()
        pltpu.async_copy(ra, acc.at[dst_v.at[c0]], sem_sa, add=True)
        pltpu.make_async_copy(table.at[src_v.at[c0 + 1]], rb, sem_gb).wait()
        pltpu.async_copy(rb, acc.at[dst_v.at[c0 + 1]], sem_sb, add=True)
        pltpu.make_async_copy(ra, acc.at[dst_v.at[c0]], sem_sa).wait()

        @pl.when(c0 + 2 < NCHUNK)
        def _():
            pltpu.async_copy(table.at[src_v.at[c0 + 2]], ra, sem_ga)

        pltpu.make_async_copy(rb, acc.at[dst_v.at[c0 + 1]], sem_sb).wait()

        @pl.when(c0 + 3 < NCHUNK)
        def _():
            pltpu.async_copy(table.at[src_v.at[c0 + 3]], rb, sem_gb)

        return 0

    lax.fori_loop(0, NCHUNK // 2, pair, 0)
    pltpu.make_async_copy(table.at[src_v.at[NCHUNK - 1]], ra, sem_ga).wait()
    pltpu.sync_copy(ra, acc.at[dst_v.at[NCHUNK - 1]], add=True)
    plsc.subcore_barrier()
    pltpu.sync_copy(acc.at[pl.ds(sid * RPT, RPT)],
                    out.at[cid, pl.ds(sid * RPT, RPT)])


def _dot(a, b):
    return jnp.dot(a, b, preferred_element_type=jnp.float32,
                   precision=lax.Precision.HIGHEST)


def _tc1_body(cnt2_ref, x_ref, w_ref, y_ref):
    cnt = cnt2_ref[0, :] + cnt2_ref[1, :]
    dinv = lax.rsqrt(cnt + 1.0)
    y_ref[...] = _dot(x_ref[...], w_ref[...]) * dinv[:, None]


def _tc2_body(agg2_ref, y_ref, cnt2_ref, b_ref, h_ref):
    cnt = cnt2_ref[0, :] + cnt2_ref[1, :]
    dinv = lax.rsqrt(cnt + 1.0)
    s = agg2_ref[0] + agg2_ref[1] + y_ref[...]
    h_ref[...] = jnp.maximum(s * dinv[:, None] + b_ref[...], 0.0)


def _gelu(v):
    return 0.5 * v * (1.0 + lax.erf(v * (1.0 / math.sqrt(2.0))))


def _tc3_body(aggs2_ref, cnt2_ref, h_ref, wl_ref, bl_ref, wr_ref,
              w1_ref, b1_ref, w2_ref, b2_ref, w3_ref, b3_ref,
              vw_ref, vb_ref, means_ref, values_ref):
    cnt = cnt2_ref[0, :] + cnt2_ref[1, :]
    inv = 1.0 / jnp.maximum(cnt, 1.0)
    mean = (aggs2_ref[0] + aggs2_ref[1]) * inv[:, None]
    h = h_ref[...]
    h2 = jnp.maximum(_dot(mean, wl_ref[...]) + bl_ref[...]
                     + _dot(h, wr_ref[...]), 0.0)
    a = _gelu(_dot(h2, w1_ref[...]) + b1_ref[...])
    a = _gelu(_dot(a, w2_ref[...]) + b2_ref[...])
    means_ref[...] = _dot(a, w3_ref[...]) + b3_ref[...]
    values_ref[...] = _dot(h2, vw_ref[...]) + vb_ref[...]


def _tc1(cnt2, x, gcn_w):
    return pl.pallas_call(
        _tc1_body,
        out_shape=jax.ShapeDtypeStruct((N, H), jnp.float32),
    )(cnt2, x, gcn_w)


def _tc2(agg2, y, cnt2, gcn_b):
    return pl.pallas_call(
        _tc2_body,
        out_shape=jax.ShapeDtypeStruct((N, H), jnp.float32),
    )(agg2, y, cnt2, gcn_b)


def _tc3(aggs2, cnt2, h, sage_wl, sage_bl, sage_wr, w1, b1, w2, b2, w3, b3,
         vw, vb):
    return pl.pallas_call(
        _tc3_body,
        out_shape=(jax.ShapeDtypeStruct((N, OUT), jnp.float32),
                   jax.ShapeDtypeStruct((N, 1), jnp.float32)),
        compiler_params=pltpu.CompilerParams(vmem_limit_bytes=64 * 1024 * 1024),
    )(aggs2, cnt2, h, sage_wl, sage_bl, sage_wr, w1, b1, w2, b2, w3, b3,
      vw, vb)


def kernel(x, edge_index, gcn_w, gcn_b, sage_wl, sage_bl, sage_wr,
           w1, b1, w2, b2, w3, b3, vw, vb):
    src2d = edge_index[0].reshape(E // CH, CH)
    dst2d = edge_index[1].reshape(E // CH, CH)
    cnt2 = _sc_count(dst2d)
    y = _tc1(cnt2, x, gcn_w)
    agg2 = _sc_agg(y, src2d, dst2d)
    h = _tc2(agg2, y, cnt2, gcn_b.reshape(1, H))
    aggs2 = _sc_agg(h, src2d, dst2d)
    means, values = _tc3(aggs2, cnt2, h, sage_wl, sage_bl.reshape(1, H),
                         sage_wr, w1, b1.reshape(1, H), w2, b2.reshape(1, H),
                         w3, b3.reshape(1, OUT), vw, vb.reshape(1, 1))
    return means, values.reshape(N)


# speedup vs baseline: 24.1123x; 1.1994x over previous
import functools
import math

import jax
import jax.numpy as jnp
from jax import lax
from jax.experimental import pallas as pl
from jax.experimental.pallas import tpu as pltpu
from jax.experimental.pallas import tpu_sc as plsc

N = 10000
E = 320000
H = 128
OUT = 8

NC = 2
NS = 16
NW = NC * NS
EPT = E // NW
CH = 80
NCHUNK = EPT // CH
RPT = N // NS
ZR = 25

_mesh = plsc.VectorSubcoreMesh(core_axis_name="c", subcore_axis_name="s")


def _zero_f32_ref(ref, nwords):
    z = jnp.zeros((16,), jnp.float32)

    def body(i, _):
        ref[pl.ds(i * 16, 16)] = z
        return 0

    lax.fori_loop(0, nwords // 16, body, 0)


def _zero_f32_ref2d(ref, rows, cols):
    z = jnp.zeros((16,), jnp.float32)

    def body(i, _):
        for j in range(cols // 16):
            ref[i, pl.ds(j * 16, 16)] = z
        return 0

    lax.fori_loop(0, rows, body, 0)


@functools.partial(
    pl.kernel,
    out_type=jax.ShapeDtypeStruct((NC, N), jnp.float32),
    mesh=_mesh,
    scratch_types=[
        pltpu.VMEM((NCHUNK, CH), jnp.int32),
        pltpu.VMEM((CH,), jnp.float32),
        pltpu.VMEM((1008,), jnp.float32),
        pltpu.VMEM_SHARED((N,), jnp.float32),
        pltpu.SemaphoreType.DMA,
        pltpu.SemaphoreType.DMA,
    ],
    compiler_params=pltpu.CompilerParams(use_tc_tiling_on_sc=False),
)
def _sc_count(dst2d, out, dst_v, ones_v, zb, cnt_sp, sem_i, sem_s):
    cid = lax.axis_index("c")
    sid = lax.axis_index("s")
    wid = cid * NS + sid

    idx_cp = pltpu.async_copy(dst2d.at[pl.ds(wid * NCHUNK, NCHUNK)], dst_v,
                              sem_i)
    one = jnp.ones((16,), jnp.float32)
    for j in range(CH // 16):
        ones_v[pl.ds(j * 16, 16)] = one
    _zero_f32_ref(zb, 1008)

    @pl.when(sid < 10)
    def _():
        pltpu.sync_copy(zb.at[pl.ds(0, 1000)], cnt_sp.at[pl.ds(sid * 1000, 1000)])

    idx_cp.wait()
    plsc.subcore_barrier()

    G = 5

    def group(g, _):
        for j in range(G):
            pltpu.async_copy(ones_v, cnt_sp.at[dst_v.at[g * G + j]], sem_s,
                             add=True)
        for j in range(G):
            pltpu.make_async_copy(ones_v, cnt_sp.at[dst_v.at[g * G + j]],
                                  sem_s).wait()
        return 0

    lax.fori_loop(0, NCHUNK // G, group, 0)
    plsc.subcore_barrier()

    @pl.when(sid < 10)
    def _():
        pltpu.sync_copy(cnt_sp.at[pl.ds(sid * 1000, 1000)],
                        out.at[cid, pl.ds(sid * 1000, 1000)])


@functools.partial(
    pl.kernel,
    out_type=jax.ShapeDtypeStruct((NC, N, H), jnp.float32),
    mesh=_mesh,
    scratch_types=[
        pltpu.VMEM((NCHUNK, CH), jnp.int32),
        pltpu.VMEM((NCHUNK, CH), jnp.int32),
        pltpu.VMEM((CH, H), jnp.float32),
        pltpu.VMEM((CH, H), jnp.float32),
        pltpu.VMEM((CH, H), jnp.float32),
        pltpu.VMEM_SHARED((N, H), jnp.float32),
        pltpu.SemaphoreType.DMA,
        pltpu.SemaphoreType.DMA,
        pltpu.SemaphoreType.DMA,
        pltpu.SemaphoreType.DMA,
        pltpu.SemaphoreType.DMA,
        pltpu.SemaphoreType.DMA,
        pltpu.SemaphoreType.DMA,
    ],
    compiler_params=pltpu.CompilerParams(use_tc_tiling_on_sc=False),
)
def _sc_agg(table, src2d, dst2d, out, src_v, dst_v, ra, rb, rc, acc,
            sem_i, sem_ga, sem_gb, sem_gc, sem_sa, sem_sb, sem_sc):
    cid = lax.axis_index("c")
    sid = lax.axis_index("s")
    wid = cid * NS + sid
    row0 = wid * NCHUNK

    ia = pltpu.async_copy(src2d.at[pl.ds(row0, NCHUNK)], src_v, sem_i)
    ib = pltpu.async_copy(dst2d.at[pl.ds(row0, NCHUNK)], dst_v, sem_i)
    _zero_f32_ref2d(ra, CH, H)
    base_r = sid * RPT
    for j in range(RPT // CH):
        pltpu.sync_copy(ra, acc.at[pl.ds(base_r + j * CH, CH)])
    rem = RPT - (RPT // CH) * CH
    pltpu.sync_copy(ra.at[pl.ds(0, rem)],
                    acc.at[pl.ds(base_r + RPT - rem, rem)])
    ia.wait()
    ib.wait()
    pltpu.async_copy(table.at[src_v.at[0]], ra, sem_ga)
    pltpu.async_copy(table.at[src_v.at[1]], rb, sem_gb)
    pltpu.async_copy(table.at[src_v.at[2]], rc, sem_gc)
    plsc.subcore_barrier()

    bufs = ((ra, sem_ga, sem_sa), (rb, sem_gb, sem_sb), (rc, sem_gc, sem_sc))
    NB = len(bufs)

    def triple(k, _):
        c0 = NB * k
        for j, (buf, sg, ss) in enumerate(bufs):
            pltpu.make_async_copy(table.at[src_v.at[c0 + j]], buf, sg).wait()
            pltpu.async_copy(buf, acc.at[dst_v.at[c0 + j]], ss, add=True)
        for j, (buf, sg, ss) in enumerate(bufs):
            pltpu.make_async_copy(buf, acc.at[dst_v.at[c0 + j]], ss).wait()

            @pl.when(c0 + NB + j < NCHUNK)
            def _():
                pltpu.async_copy(table.at[src_v.at[c0 + NB + j]], buf, sg)

        return 0

    lax.fori_loop(0, NCHUNK // NB, triple, 0)
    for j in range(NCHUNK - (NCHUNK // NB) * NB):
        c = (NCHUNK // NB) * NB + j
        buf, sg, ss = bufs[j]
        pltpu.make_async_copy(table.at[src_v.at[c]], buf, sg).wait()
        pltpu.sync_copy(buf, acc.at[dst_v.at[c]], add=True)
    plsc.subcore_barrier()
    pltpu.sync_copy(acc.at[pl.ds(sid * RPT, RPT)],
                    out.at[cid, pl.ds(sid * RPT, RPT)])


def _dot(a, b):
    return jnp.dot(a, b, preferred_element_type=jnp.float32,
                   precision=lax.Precision.HIGHEST)


def _tc1_body(cnt2_ref, x_ref, w_ref, y_ref):
    cnt = cnt2_ref[0, :] + cnt2_ref[1, :]
    dinv = lax.rsqrt(cnt + 1.0)
    y_ref[...] = _dot(x_ref[...], w_ref[...]) * dinv[:, None]


def _tc2_body(agg2_ref, y_ref, cnt2_ref, b_ref, h_ref):
    cnt = cnt2_ref[0, :] + cnt2_ref[1, :]
    dinv = lax.rsqrt(cnt + 1.0)
    s = agg2_ref[0] + agg2_ref[1] + y_ref[...]
    h_ref[...] = jnp.maximum(s * dinv[:, None] + b_ref[...], 0.0)


def _gelu(v):
    return 0.5 * v * (1.0 + lax.erf(v * (1.0 / math.sqrt(2.0))))


def _tc3_body(aggs2_ref, cnt2_ref, h_ref, wl_ref, bl_ref, wr_ref,
              w1_ref, b1_ref, w2_ref, b2_ref, w3_ref, b3_ref,
              vw_ref, vb_ref, means_ref, values_ref):
    cnt = cnt2_ref[0, :] + cnt2_ref[1, :]
    inv = 1.0 / jnp.maximum(cnt, 1.0)
    mean = (aggs2_ref[0] + aggs2_ref[1]) * inv[:, None]
    h = h_ref[...]
    h2 = jnp.maximum(_dot(mean, wl_ref[...]) + bl_ref[...]
                     + _dot(h, wr_ref[...]), 0.0)
    a = _gelu(_dot(h2, w1_ref[...]) + b1_ref[...])
    a = _gelu(_dot(a, w2_ref[...]) + b2_ref[...])
    means_ref[...] = _dot(a, w3_ref[...]) + b3_ref[...]
    values_ref[...] = _dot(h2, vw_ref[...]) + vb_ref[...]


def _tc1(cnt2, x, gcn_w):
    return pl.pallas_call(
        _tc1_body,
        out_shape=jax.ShapeDtypeStruct((N, H), jnp.float32),
    )(cnt2, x, gcn_w)


def _tc2(agg2, y, cnt2, gcn_b):
    return pl.pallas_call(
        _tc2_body,
        out_shape=jax.ShapeDtypeStruct((N, H), jnp.float32),
    )(agg2, y, cnt2, gcn_b)


def _tc3(aggs2, cnt2, h, sage_wl, sage_bl, sage_wr, w1, b1, w2, b2, w3, b3,
         vw, vb):
    return pl.pallas_call(
        _tc3_body,
        out_shape=(jax.ShapeDtypeStruct((N, OUT), jnp.float32),
                   jax.ShapeDtypeStruct((N, 1), jnp.float32)),
        compiler_params=pltpu.CompilerParams(vmem_limit_bytes=64 * 1024 * 1024),
    )(aggs2, cnt2, h, sage_wl, sage_bl, sage_wr, w1, b1, w2, b2, w3, b3,
      vw, vb)


def kernel(x, edge_index, gcn_w, gcn_b, sage_wl, sage_bl, sage_wr,
           w1, b1, w2, b2, w3, b3, vw, vb):
    src2d = edge_index[0].reshape(E // CH, CH)
    dst2d = edge_index[1].reshape(E // CH, CH)
    cnt2 = _sc_count(dst2d)
    y = _tc1(cnt2, x, gcn_w)
    agg2 = _sc_agg(y, src2d, dst2d)
    h = _tc2(agg2, y, cnt2, gcn_b.reshape(1, H))
    aggs2 = _sc_agg(h, src2d, dst2d)
    means, values = _tc3(aggs2, cnt2, h, sage_wl, sage_bl.reshape(1, H),
                         sage_wr, w1, b1.reshape(1, H), w2, b2.reshape(1, H),
                         w3, b3.reshape(1, OUT), vw, vb.reshape(1, 1))
    return means, values.reshape(N)


# speedup vs baseline: 25.1757x; 1.0441x over previous
import functools
import math

import jax
import jax.numpy as jnp
from jax import lax
from jax.experimental import pallas as pl
from jax.experimental.pallas import tpu as pltpu
from jax.experimental.pallas import tpu_sc as plsc

N = 10000
E = 320000
H = 128
OUT = 8

NC = 2
NS = 16
NW = NC * NS
EPT = E // NW
CH = 80
NCHUNK = EPT // CH
CHA = 40
NCHUNKA = EPT // CHA
NB = 5
RPT = N // NS

_mesh = plsc.VectorSubcoreMesh(core_axis_name="c", subcore_axis_name="s")


def _zero_f32_ref(ref, nwords):
    z = jnp.zeros((16,), jnp.float32)

    def body(i, _):
        ref[pl.ds(i * 16, 16)] = z
        return 0

    lax.fori_loop(0, nwords // 16, body, 0)


def _zero_f32_ref2d(ref, rows, cols):
    z = jnp.zeros((16,), jnp.float32)

    def body(i, _):
        for j in range(cols // 16):
            ref[i, pl.ds(j * 16, 16)] = z
        return 0

    lax.fori_loop(0, rows, body, 0)


@functools.partial(
    pl.kernel,
    out_type=jax.ShapeDtypeStruct((NC, N), jnp.float32),
    mesh=_mesh,
    scratch_types=[
        pltpu.VMEM((NCHUNK, CH), jnp.int32),
        pltpu.VMEM((CH,), jnp.float32),
        pltpu.VMEM((1008,), jnp.float32),
        pltpu.VMEM_SHARED((N,), jnp.float32),
        pltpu.SemaphoreType.DMA,
        pltpu.SemaphoreType.DMA,
    ],
    compiler_params=pltpu.CompilerParams(use_tc_tiling_on_sc=False),
)
def _sc_count(dst2d, out, dst_v, ones_v, zb, cnt_sp, sem_i, sem_s):
    cid = lax.axis_index("c")
    sid = lax.axis_index("s")
    wid = cid * NS + sid

    idx_cp = pltpu.async_copy(dst2d.at[pl.ds(wid * NCHUNK, NCHUNK)], dst_v,
                              sem_i)
    one = jnp.ones((16,), jnp.float32)
    for j in range(CH // 16):
        ones_v[pl.ds(j * 16, 16)] = one
    _zero_f32_ref(zb, 1008)

    @pl.when(sid < 10)
    def _():
        pltpu.sync_copy(zb.at[pl.ds(0, 1000)], cnt_sp.at[pl.ds(sid * 1000, 1000)])

    idx_cp.wait()
    plsc.subcore_barrier()

    G = 5

    def group(g, _):
        for j in range(G):
            pltpu.async_copy(ones_v, cnt_sp.at[dst_v.at[g * G + j]], sem_s,
                             add=True)
        for j in range(G):
            pltpu.make_async_copy(ones_v, cnt_sp.at[dst_v.at[g * G + j]],
                                  sem_s).wait()
        return 0

    lax.fori_loop(0, NCHUNK // G, group, 0)
    plsc.subcore_barrier()

    @pl.when(sid < 10)
    def _():
        pltpu.sync_copy(cnt_sp.at[pl.ds(sid * 1000, 1000)],
                        out.at[cid, pl.ds(sid * 1000, 1000)])


@functools.partial(
    pl.kernel,
    out_type=jax.ShapeDtypeStruct((NC, N, H), jnp.float32),
    mesh=_mesh,
    scratch_types=(
        [
            pltpu.VMEM((NCHUNKA, CHA), jnp.int32),
            pltpu.VMEM((NCHUNKA, CHA), jnp.int32),
        ]
        + [pltpu.VMEM((CHA, H), jnp.float32) for _ in range(NB)]
        + [pltpu.VMEM_SHARED((N, H), jnp.float32)]
        + [pltpu.SemaphoreType.DMA for _ in range(2 * NB + 1)]
    ),
    compiler_params=pltpu.CompilerParams(use_tc_tiling_on_sc=False),
)
def _sc_agg(table, src2d, dst2d, out, src_v, dst_v, *rest):
    rows = rest[:NB]
    acc = rest[NB]
    sem_i = rest[NB + 1]
    gsems = rest[NB + 2:NB + 2 + NB]
    ssems = rest[NB + 2 + NB:]
    cid = lax.axis_index("c")
    sid = lax.axis_index("s")
    wid = cid * NS + sid
    row0 = wid * NCHUNKA

    ia = pltpu.async_copy(src2d.at[pl.ds(row0, NCHUNKA)], src_v, sem_i)
    ib = pltpu.async_copy(dst2d.at[pl.ds(row0, NCHUNKA)], dst_v, sem_i)
    _zero_f32_ref2d(rows[0], CHA, H)
    base_r = sid * RPT
    for j in range(RPT // CHA):
        pltpu.sync_copy(rows[0], acc.at[pl.ds(base_r + j * CHA, CHA)])
    rem = RPT - (RPT // CHA) * CHA
    if rem:
        pltpu.sync_copy(rows[0].at[pl.ds(0, rem)],
                        acc.at[pl.ds(base_r + RPT - rem, rem)])
    ia.wait()
    ib.wait()
    for j in range(NB):
        pltpu.async_copy(table.at[src_v.at[j]], rows[j], gsems[j])
    plsc.subcore_barrier()

    def round_(k, _):
        c0 = NB * k
        for j in range(NB):
            pltpu.make_async_copy(table.at[src_v.at[c0 + j]], rows[j],
                                  gsems[j]).wait()
            pltpu.async_copy(rows[j], acc.at[dst_v.at[c0 + j]], ssems[j],
                             add=True)
        for j in range(NB):
            pltpu.make_async_copy(rows[j], acc.at[dst_v.at[c0 + j]],
                                  ssems[j]).wait()

            @pl.when(c0 + NB + j < NCHUNKA)
            def _():
                pltpu.async_copy(table.at[src_v.at[c0 + NB + j]], rows[j],
                                 gsems[j])

        return 0

    lax.fori_loop(0, NCHUNKA // NB, round_, 0)
    plsc.subcore_barrier()
    pltpu.sync_copy(acc.at[pl.ds(sid * RPT, RPT)],
                    out.at[cid, pl.ds(sid * RPT, RPT)])


def _dot(a, b):
    return jnp.dot(a, b, preferred_element_type=jnp.float32,
                   precision=lax.Precision.HIGHEST)


def _tc1_body(cnt2_ref, x_ref, w_ref, y_ref):
    cnt = cnt2_ref[0, :] + cnt2_ref[1, :]
    dinv = lax.rsqrt(cnt + 1.0)
    y_ref[...] = _dot(x_ref[...], w_ref[...]) * dinv[:, None]


def _tc2_body(agg2_ref, y_ref, cnt2_ref, b_ref, h_ref):
    cnt = cnt2_ref[0, :] + cnt2_ref[1, :]
    dinv = lax.rsqrt(cnt + 1.0)
    s = agg2_ref[0] + agg2_ref[1] + y_ref[...]
    h_ref[...] = jnp.maximum(s * dinv[:, None] + b_ref[...], 0.0)


def _gelu(v):
    return 0.5 * v * (1.0 + lax.erf(v * (1.0 / math.sqrt(2.0))))


def _tc3_body(aggs2_ref, cnt2_ref, h_ref, wl_ref, bl_ref, wr_ref,
              w1_ref, b1_ref, w2_ref, b2_ref, w3_ref, b3_ref,
              vw_ref, vb_ref, means_ref, values_ref):
    cnt = cnt2_ref[0, :] + cnt2_ref[1, :]
    inv = 1.0 / jnp.maximum(cnt, 1.0)
    mean = (aggs2_ref[0] + aggs2_ref[1]) * inv[:, None]
    h = h_ref[...]
    h2 = jnp.maximum(_dot(mean, wl_ref[...]) + bl_ref[...]
                     + _dot(h, wr_ref[...]), 0.0)
    a = _gelu(_dot(h2, w1_ref[...]) + b1_ref[...])
    a = _gelu(_dot(a, w2_ref[...]) + b2_ref[...])
    means_ref[...] = _dot(a, w3_ref[...]) + b3_ref[...]
    values_ref[...] = _dot(h2, vw_ref[...]) + vb_ref[...]


def _tc1(cnt2, x, gcn_w):
    return pl.pallas_call(
        _tc1_body,
        out_shape=jax.ShapeDtypeStruct((N, H), jnp.float32),
    )(cnt2, x, gcn_w)


def _tc2(agg2, y, cnt2, gcn_b):
    return pl.pallas_call(
        _tc2_body,
        out_shape=jax.ShapeDtypeStruct((N, H), jnp.float32),
    )(agg2, y, cnt2, gcn_b)


def _tc3(aggs2, cnt2, h, sage_wl, sage_bl, sage_wr, w1, b1, w2, b2, w3, b3,
         vw, vb):
    return pl.pallas_call(
        _tc3_body,
        out_shape=(jax.ShapeDtypeStruct((N, OUT), jnp.float32),
                   jax.ShapeDtypeStruct((N, 1), jnp.float32)),
        compiler_params=pltpu.CompilerParams(vmem_limit_bytes=64 * 1024 * 1024),
    )(aggs2, cnt2, h, sage_wl, sage_bl, sage_wr, w1, b1, w2, b2, w3, b3,
      vw, vb)


def kernel(x, edge_index, gcn_w, gcn_b, sage_wl, sage_bl, sage_wr,
           w1, b1, w2, b2, w3, b3, vw, vb):
    src2d = edge_index[0].reshape(E // CHA, CHA)
    dst2d = edge_index[1].reshape(E // CHA, CHA)
    dst2dc = edge_index[1].reshape(E // CH, CH)
    cnt2 = _sc_count(dst2dc)
    y = _tc1(cnt2, x, gcn_w)
    agg2 = _sc_agg(y, src2d, dst2d)
    h = _tc2(agg2, y, cnt2, gcn_b.reshape(1, H))
    aggs2 = _sc_agg(h, src2d, dst2d)
    means, values = _tc3(aggs2, cnt2, h, sage_wl, sage_bl.reshape(1, H),
                         sage_wr, w1, b1.reshape(1, H), w2, b2.reshape(1, H),
                         w3, b3.reshape(1, OUT), vw, vb.reshape(1, 1))
    return means, values.reshape(N)


# speedup vs baseline: 28.4582x; 1.1304x over previous
import functools
import math

import jax
import jax.numpy as jnp
from jax import lax
from jax.experimental import pallas as pl
from jax.experimental.pallas import tpu as pltpu
from jax.experimental.pallas import tpu_sc as plsc

N = 10000
E = 320000
H = 128
OUT = 8

NC = 2
NS = 16
NW = NC * NS
EPT = E // NW
CH = 80
NCHUNK = EPT // CH
CHA = 40
NCHUNKA = EPT // CHA
NB = 5
RPT = N // NS

_mesh = plsc.VectorSubcoreMesh(core_axis_name="c", subcore_axis_name="s")


def _zero_f32_ref(ref, nwords):
    z = jnp.zeros((16,), jnp.float32)

    def body(i, _):
        ref[pl.ds(i * 16, 16)] = z
        return 0

    lax.fori_loop(0, nwords // 16, body, 0)


def _zero_f32_ref2d(ref, rows, cols):
    z = jnp.zeros((16,), jnp.float32)

    def body(i, _):
        for j in range(cols // 16):
            ref[i, pl.ds(j * 16, 16)] = z
        return 0

    lax.fori_loop(0, rows, body, 0)


@functools.partial(
    pl.kernel,
    out_type=jax.ShapeDtypeStruct((NC, N), jnp.float32),
    mesh=_mesh,
    scratch_types=[
        pltpu.VMEM((NCHUNK, CH), jnp.int32),
        pltpu.VMEM((CH,), jnp.float32),
        pltpu.VMEM((1008,), jnp.float32),
        pltpu.VMEM_SHARED((N,), jnp.float32),
        pltpu.SemaphoreType.DMA,
        pltpu.SemaphoreType.DMA,
    ],
    compiler_params=pltpu.CompilerParams(use_tc_tiling_on_sc=False),
)
def _sc_count(dst2d, out, dst_v, ones_v, zb, cnt_sp, sem_i, sem_s):
    cid = lax.axis_index("c")
    sid = lax.axis_index("s")
    wid = cid * NS + sid

    idx_cp = pltpu.async_copy(dst2d.at[pl.ds(wid * NCHUNK, NCHUNK)], dst_v,
                              sem_i)
    one = jnp.ones((16,), jnp.float32)
    for j in range(CH // 16):
        ones_v[pl.ds(j * 16, 16)] = one
    _zero_f32_ref(zb, 1008)

    @pl.when(sid < 10)
    def _():
        pltpu.sync_copy(zb.at[pl.ds(0, 1000)], cnt_sp.at[pl.ds(sid * 1000, 1000)])

    idx_cp.wait()
    plsc.subcore_barrier()

    G = 5

    def group(g, _):
        for j in range(G):
            pltpu.async_copy(ones_v, cnt_sp.at[dst_v.at[g * G + j]], sem_s,
                             add=True)
        for j in range(G):
            pltpu.make_async_copy(ones_v, cnt_sp.at[dst_v.at[g * G + j]],
                                  sem_s).wait()
        return 0

    lax.fori_loop(0, NCHUNK // G, group, 0)
    plsc.subcore_barrier()

    @pl.when(sid < 10)
    def _():
        pltpu.sync_copy(cnt_sp.at[pl.ds(sid * 1000, 1000)],
                        out.at[cid, pl.ds(sid * 1000, 1000)])


@functools.partial(
    pl.kernel,
    out_type=jax.ShapeDtypeStruct((NC, N, H), jnp.float32),
    mesh=_mesh,
    scratch_types=(
        [
            pltpu.VMEM((NCHUNKA, CHA), jnp.int32),
            pltpu.VMEM((NCHUNKA, CHA), jnp.int32),
        ]
        + [pltpu.VMEM((CHA, H), jnp.float32) for _ in range(NB)]
        + [pltpu.VMEM_SHARED((N, H), jnp.float32)]
        + [pltpu.SemaphoreType.DMA for _ in range(2 * NB + 1)]
    ),
    compiler_params=pltpu.CompilerParams(use_tc_tiling_on_sc=False),
)
def _sc_agg(table, src2d, dst2d, out, src_v, dst_v, *rest):
    rows = rest[:NB]
    acc = rest[NB]
    sem_i = rest[NB + 1]
    gsems = rest[NB + 2:NB + 2 + NB]
    ssems = rest[NB + 2 + NB:]
    cid = lax.axis_index("c")
    sid = lax.axis_index("s")
    wid = cid * NS + sid
    row0 = wid * NCHUNKA

    ia = pltpu.async_copy(src2d.at[pl.ds(row0, NCHUNKA)], src_v, sem_i)
    ib = pltpu.async_copy(dst2d.at[pl.ds(row0, NCHUNKA)], dst_v, sem_i)
    _zero_f32_ref2d(rows[0], CHA, H)
    base_r = sid * RPT
    for j in range(RPT // CHA):
        pltpu.sync_copy(rows[0], acc.at[pl.ds(base_r + j * CHA, CHA)])
    rem = RPT - (RPT // CHA) * CHA
    if rem:
        pltpu.sync_copy(rows[0].at[pl.ds(0, rem)],
                        acc.at[pl.ds(base_r + RPT - rem, rem)])
    ia.wait()
    ib.wait()
    for j in range(NB):
        pltpu.async_copy(table.at[src_v.at[j]], rows[j], gsems[j])
    plsc.subcore_barrier()

    def round_(k, _):
        c0 = NB * k
        for j in range(NB):
            pltpu.make_async_copy(table.at[src_v.at[c0 + j]], rows[j],
                                  gsems[j]).wait()
            pltpu.async_copy(rows[j], acc.at[dst_v.at[c0 + j]], ssems[j],
                             add=True)
        for j in range(NB):
            pltpu.make_async_copy(rows[j], acc.at[dst_v.at[c0 + j]],
                                  ssems[j]).wait()

            @pl.when(c0 + NB + j < NCHUNKA)
            def _():
                pltpu.async_copy(table.at[src_v.at[c0 + NB + j]], rows[j],
                                 gsems[j])

        return 0

    lax.fori_loop(0, NCHUNKA // NB, round_, 0)
    plsc.subcore_barrier()
    pltpu.sync_copy(acc.at[pl.ds(sid * RPT, RPT)],
                    out.at[cid, pl.ds(sid * RPT, RPT)])


def _dot(a, b):
    return jnp.dot(a, b, preferred_element_type=jnp.float32,
                   precision=lax.Precision.DEFAULT)


def _tc1_body(cnt2_ref, x_ref, w_ref, y_ref):
    cnt = cnt2_ref[0, :] + cnt2_ref[1, :]
    dinv = lax.rsqrt(cnt + 1.0)
    y_ref[...] = _dot(x_ref[...], w_ref[...]) * dinv[:, None]


def _tc2_body(agg2_ref, y_ref, cnt2_ref, b_ref, h_ref):
    cnt = cnt2_ref[0, :] + cnt2_ref[1, :]
    dinv = lax.rsqrt(cnt + 1.0)
    s = agg2_ref[0] + agg2_ref[1] + y_ref[...]
    h_ref[...] = jnp.maximum(s * dinv[:, None] + b_ref[...], 0.0)


def _gelu(v):
    return 0.5 * v * (1.0 + lax.erf(v * (1.0 / math.sqrt(2.0))))


def _tc3_body(aggs2_ref, cnt2_ref, h_ref, wl_ref, bl_ref, wr_ref,
              w1_ref, b1_ref, w2_ref, b2_ref, w3_ref, b3_ref,
              vw_ref, vb_ref, means_ref, values_ref):
    cnt = cnt2_ref[0, :] + cnt2_ref[1, :]
    inv = 1.0 / jnp.maximum(cnt, 1.0)
    mean = (aggs2_ref[0] + aggs2_ref[1]) * inv[:, None]
    h = h_ref[...]
    h2 = jnp.maximum(_dot(mean, wl_ref[...]) + bl_ref[...]
                     + _dot(h, wr_ref[...]), 0.0)
    a = _gelu(_dot(h2, w1_ref[...]) + b1_ref[...])
    a = _gelu(_dot(a, w2_ref[...]) + b2_ref[...])
    means_ref[...] = _dot(a, w3_ref[...]) + b3_ref[...]
    values_ref[...] = _dot(h2, vw_ref[...]) + vb_ref[...]


def _tc1(cnt2, x, gcn_w):
    return pl.pallas_call(
        _tc1_body,
        out_shape=jax.ShapeDtypeStruct((N, H), jnp.float32),
    )(cnt2, x, gcn_w)


def _tc2(agg2, y, cnt2, gcn_b):
    return pl.pallas_call(
        _tc2_body,
        out_shape=jax.ShapeDtypeStruct((N, H), jnp.float32),
    )(agg2, y, cnt2, gcn_b)


def _tc3(aggs2, cnt2, h, sage_wl, sage_bl, sage_wr, w1, b1, w2, b2, w3, b3,
         vw, vb):
    return pl.pallas_call(
        _tc3_body,
        out_shape=(jax.ShapeDtypeStruct((N, OUT), jnp.float32),
                   jax.ShapeDtypeStruct((N, 1), jnp.float32)),
        compiler_params=pltpu.CompilerParams(vmem_limit_bytes=64 * 1024 * 1024),
    )(aggs2, cnt2, h, sage_wl, sage_bl, sage_wr, w1, b1, w2, b2, w3, b3,
      vw, vb)


def kernel(x, edge_index, gcn_w, gcn_b, sage_wl, sage_bl, sage_wr,
           w1, b1, w2, b2, w3, b3, vw, vb):
    src2d = edge_index[0].reshape(E // CHA, CHA)
    dst2d = edge_index[1].reshape(E // CHA, CHA)
    dst2dc = edge_index[1].reshape(E // CH, CH)
    cnt2 = _sc_count(dst2dc)
    y = _tc1(cnt2, x, gcn_w)
    agg2 = _sc_agg(y, src2d, dst2d)
    h = _tc2(agg2, y, cnt2, gcn_b.reshape(1, H))
    aggs2 = _sc_agg(h, src2d, dst2d)
    means, values = _tc3(aggs2, cnt2, h, sage_wl, sage_bl.reshape(1, H),
                         sage_wr, w1, b1.reshape(1, H), w2, b2.reshape(1, H),
                         w3, b3.reshape(1, OUT), vw, vb.reshape(1, 1))
    return means, values.reshape(N)


# speedup vs baseline: 28.5470x; 1.0031x over previous
import functools
import math

import jax
import jax.numpy as jnp
from jax import lax
from jax.experimental import pallas as pl
from jax.experimental.pallas import tpu as pltpu
from jax.experimental.pallas import tpu_sc as plsc

N = 10000
E = 320000
H = 128
OUT = 8

NC = 2
NS = 16
NW = NC * NS
EPT = E // NW
CH = 80
NCHUNK = EPT // CH
CHA = 40
NCHUNKA = EPT // CHA
NB = 5
RPT = N // NS

_mesh = plsc.VectorSubcoreMesh(core_axis_name="c", subcore_axis_name="s")


def _zero_f32_ref(ref, nwords):
    z = jnp.zeros((16,), jnp.float32)

    def body(i, _):
        ref[pl.ds(i * 16, 16)] = z
        return 0

    lax.fori_loop(0, nwords // 16, body, 0)


def _zero_f32_ref2d(ref, rows, cols):
    z = jnp.zeros((16,), jnp.float32)

    def body(i, _):
        for j in range(cols // 16):
            ref[i, pl.ds(j * 16, 16)] = z
        return 0

    lax.fori_loop(0, rows, body, 0)


@functools.partial(
    pl.kernel,
    out_type=jax.ShapeDtypeStruct((NC, N), jnp.float32),
    mesh=_mesh,
    scratch_types=[
        pltpu.VMEM((NCHUNK, CH), jnp.int32),
        pltpu.VMEM((CH,), jnp.float32),
        pltpu.VMEM((1008,), jnp.float32),
        pltpu.VMEM_SHARED((N,), jnp.float32),
        pltpu.SemaphoreType.DMA,
        pltpu.SemaphoreType.DMA,
    ],
    compiler_params=pltpu.CompilerParams(use_tc_tiling_on_sc=False),
)
def _sc_count(dst2d, out, dst_v, ones_v, zb, cnt_sp, sem_i, sem_s):
    cid = lax.axis_index("c")
    sid = lax.axis_index("s")
    wid = cid * NS + sid

    idx_cp = pltpu.async_copy(dst2d.at[pl.ds(wid * NCHUNK, NCHUNK)], dst_v,
                              sem_i)
    one = jnp.ones((16,), jnp.float32)
    for j in range(CH // 16):
        ones_v[pl.ds(j * 16, 16)] = one
    _zero_f32_ref(zb, 1008)

    @pl.when(sid < 10)
    def _():
        pltpu.sync_copy(zb.at[pl.ds(0, 1000)], cnt_sp.at[pl.ds(sid * 1000, 1000)])

    idx_cp.wait()
    plsc.subcore_barrier()

    G = 5

    def group(g, _):
        for j in range(G):
            pltpu.async_copy(ones_v, cnt_sp.at[dst_v.at[g * G + j]], sem_s,
                             add=True)
        for j in range(G):
            pltpu.make_async_copy(ones_v, cnt_sp.at[dst_v.at[g * G + j]],
                                  sem_s).wait()
        return 0

    lax.fori_loop(0, NCHUNK // G, group, 0)
    plsc.subcore_barrier()

    @pl.when(sid < 10)
    def _():
        pltpu.sync_copy(cnt_sp.at[pl.ds(sid * 1000, 1000)],
                        out.at[cid, pl.ds(sid * 1000, 1000)])


@functools.partial(
    pl.kernel,
    out_type=jax.ShapeDtypeStruct((NC, N, H), jnp.float32),
    mesh=_mesh,
    scratch_types=(
        [
            pltpu.VMEM((NCHUNKA, CHA), jnp.int32),
            pltpu.VMEM((NCHUNKA, CHA), jnp.int32),
        ]
        + [pltpu.VMEM((CHA, H), jnp.float32) for _ in range(NB)]
        + [pltpu.VMEM_SHARED((N, H), jnp.float32)]
        + [pltpu.SemaphoreType.DMA for _ in range(2 * NB + 1)]
    ),
    compiler_params=pltpu.CompilerParams(use_tc_tiling_on_sc=False),
)
def _sc_agg(table, src2d, dst2d, out, src_v, dst_v, *rest):
    rows = rest[:NB]
    acc = rest[NB]
    sem_i = rest[NB + 1]
    gsems = rest[NB + 2:NB + 2 + NB]
    ssems = rest[NB + 2 + NB:]
    cid = lax.axis_index("c")
    sid = lax.axis_index("s")
    wid = cid * NS + sid
    row0 = wid * NCHUNKA

    ia = pltpu.async_copy(src2d.at[pl.ds(row0, NCHUNKA)], src_v, sem_i)
    ib = pltpu.async_copy(dst2d.at[pl.ds(row0, NCHUNKA)], dst_v, sem_i)
    _zero_f32_ref2d(rows[0], CHA, H)
    base_r = sid * RPT
    nz = RPT // CHA
    for j in range(nz):
        pltpu.async_copy(rows[0], acc.at[pl.ds(base_r + j * CHA, CHA)],
                         gsems[0])
    rem = RPT - nz * CHA
    if rem:
        pltpu.sync_copy(rows[0].at[pl.ds(0, rem)],
                        acc.at[pl.ds(base_r + RPT - rem, rem)])
    for j in range(nz):
        pltpu.make_async_copy(rows[0], acc.at[pl.ds(base_r + j * CHA, CHA)],
                              gsems[0]).wait()
    ia.wait()
    ib.wait()
    for j in range(NB):
        pltpu.async_copy(table.at[src_v.at[j]], rows[j], gsems[j])
    plsc.subcore_barrier()

    def round_(k, _):
        c0 = NB * k
        for j in range(NB):
            pltpu.make_async_copy(table.at[src_v.at[c0 + j]], rows[j],
                                  gsems[j]).wait()
            pltpu.async_copy(rows[j], acc.at[dst_v.at[c0 + j]], ssems[j],
                             add=True)
        for j in range(NB):
            pltpu.make_async_copy(rows[j], acc.at[dst_v.at[c0 + j]],
                                  ssems[j]).wait()

            @pl.when(c0 + NB + j < NCHUNKA)
            def _():
                pltpu.async_copy(table.at[src_v.at[c0 + NB + j]], rows[j],
                                 gsems[j])

        return 0

    lax.fori_loop(0, NCHUNKA // NB, round_, 0)
    plsc.subcore_barrier()
    pltpu.sync_copy(acc.at[pl.ds(sid * RPT, RPT)],
                    out.at[cid, pl.ds(sid * RPT, RPT)])


def _dot(a, b):
    return jnp.dot(a, b, preferred_element_type=jnp.float32,
                   precision=lax.Precision.DEFAULT)


def _tc1_body(cnt2_ref, x_ref, w_ref, y_ref):
    cnt = cnt2_ref[0, :] + cnt2_ref[1, :]
    dinv = lax.rsqrt(cnt + 1.0)
    y_ref[...] = _dot(x_ref[...], w_ref[...]) * dinv[:, None]


def _tc2_body(agg2_ref, y_ref, cnt2_ref, b_ref, h_ref):
    cnt = cnt2_ref[0, :] + cnt2_ref[1, :]
    dinv = lax.rsqrt(cnt + 1.0)
    s = agg2_ref[0] + agg2_ref[1] + y_ref[...]
    h_ref[...] = jnp.maximum(s * dinv[:, None] + b_ref[...], 0.0)


def _gelu(v):
    return 0.5 * v * (1.0 + lax.erf(v * (1.0 / math.sqrt(2.0))))


def _tc3_body(aggs2_ref, cnt2_ref, h_ref, wl_ref, bl_ref, wr_ref,
              w1_ref, b1_ref, w2_ref, b2_ref, w3_ref, b3_ref,
              vw_ref, vb_ref, means_ref, values_ref):
    cnt = cnt2_ref[0, :] + cnt2_ref[1, :]
    inv = 1.0 / jnp.maximum(cnt, 1.0)
    mean = (aggs2_ref[0] + aggs2_ref[1]) * inv[:, None]
    h = h_ref[...]
    h2 = jnp.maximum(_dot(mean, wl_ref[...]) + bl_ref[...]
                     + _dot(h, wr_ref[...]), 0.0)
    a = _gelu(_dot(h2, w1_ref[...]) + b1_ref[...])
    a = _gelu(_dot(a, w2_ref[...]) + b2_ref[...])
    means_ref[...] = _dot(a, w3_ref[...]) + b3_ref[...]
    values_ref[...] = _dot(h2, vw_ref[...]) + vb_ref[...]


def _tc1(cnt2, x, gcn_w):
    return pl.pallas_call(
        _tc1_body,
        out_shape=jax.ShapeDtypeStruct((N, H), jnp.float32),
    )(cnt2, x, gcn_w)


def _tc2(agg2, y, cnt2, gcn_b):
    return pl.pallas_call(
        _tc2_body,
        out_shape=jax.ShapeDtypeStruct((N, H), jnp.float32),
    )(agg2, y, cnt2, gcn_b)


def _tc3(aggs2, cnt2, h, sage_wl, sage_bl, sage_wr, w1, b1, w2, b2, w3, b3,
         vw, vb):
    return pl.pallas_call(
        _tc3_body,
        out_shape=(jax.ShapeDtypeStruct((N, OUT), jnp.float32),
                   jax.ShapeDtypeStruct((N, 1), jnp.float32)),
        compiler_params=pltpu.CompilerParams(vmem_limit_bytes=64 * 1024 * 1024),
    )(aggs2, cnt2, h, sage_wl, sage_bl, sage_wr, w1, b1, w2, b2, w3, b3,
      vw, vb)


def kernel(x, edge_index, gcn_w, gcn_b, sage_wl, sage_bl, sage_wr,
           w1, b1, w2, b2, w3, b3, vw, vb):
    src2d = edge_index[0].reshape(E // CHA, CHA)
    dst2d = edge_index[1].reshape(E // CHA, CHA)
    dst2dc = edge_index[1].reshape(E // CH, CH)
    cnt2 = _sc_count(dst2dc)
    y = _tc1(cnt2, x, gcn_w)
    agg2 = _sc_agg(y, src2d, dst2d)
    h = _tc2(agg2, y, cnt2, gcn_b.reshape(1, H))
    aggs2 = _sc_agg(h, src2d, dst2d)
    means, values = _tc3(aggs2, cnt2, h, sage_wl, sage_bl.reshape(1, H),
                         sage_wr, w1, b1.reshape(1, H), w2, b2.reshape(1, H),
                         w3, b3.reshape(1, OUT), vw, vb.reshape(1, 1))
    return means, values.reshape(N)
